# baseline (speedup 1.0000x reference)
import jax
import jax.numpy as jnp
from jax import lax
from jax.experimental import pallas as pl
from jax.experimental.pallas import tpu as pltpu
from jax.experimental.pallas import tpu_sc as plsc

M, J, I, L, W = 16, 2, 3, 2, 9

_B_G = W * L
_B_T = _B_G + J * L
_B_H = _B_T + J * L
_B_W = _B_H + J * I * L
_B_B = _B_W + L * I


def _pack_body(con, gam, hw, tw, o):
    def put(r, v):
        o[r % 8, pl.ds((r // 8) * 16, 16)] = v

    for j in range(J):
        for l in range(L):
            put(_B_G + j * L + l, gam[:, j + 1, l])
            put(_B_T + j * L + l, con[:, j, l])
    for j in range(J):
        for i in range(I):
            for l in range(L):
                put(_B_H + (j * I + i) * L + l, hw[:, j, i, l])
    for l in range(L):
        for i in range(I):
            put(_B_W + l * I + i, tw[:, l, i])


def _bcast_lane(v, k):
    return v.at[jnp.full((16,), k, jnp.int32)].get(mode="promise_in_bounds")


def _body(a_hbm, st_hbm, tb_hbm, out_hbm, a_v, st_v, tb_v, o_v, sem):
    @pl.when((lax.axis_index("c") == 0) & (lax.axis_index("s") == 0))
    def _():
        cps = [pltpu.async_copy(src, dst, sem) for src, dst in
               [(a_hbm, a_v), (st_hbm, st_v), (tb_hbm, tb_v)]]
        for cp in cps:
            cp.wait()

        def row(r):
            return a_v[r % 8, pl.ds((r // 8) * 16, 16)]

        s_lo = st_v[0, pl.ds(0, 16)]
        s_hi = st_v[0, pl.ds(2, 16)]
        s = [[_bcast_lane(s_lo, w * L + l) if w * L + l < 16
              else _bcast_lane(s_hi, w * L + l - 2)
              for l in range(L)] for w in range(W)]
        cap = [jnp.zeros((16,), jnp.float32) for _ in range(I)]
        for j in range(J):
            gam = [row(_B_G + j * L + l) for l in range(L)]
            sig = [1.0 / (1.0 + jnp.exp(-10.0 * (g - 0.5))) for g in gam]
            tem = [row(_B_T + j * L + l) for l in range(L)]
            best_q = None
            sel = [None] * L
            for w in range(W):
                d0 = tem[0] - s[w][0]
                q = sig[0] * d0 * d0
                for l in range(1, L):
                    dl = tem[l] - s[w][l]
                    q = q + sig[l] * dl * dl
                if best_q is None:
                    best_q = q
                    sel = list(s[w])
                else:
                    take = q < best_q
                    best_q = jnp.where(take, q, best_q)
                    sel = [jnp.where(take, s[w][l], sel[l]) for l in range(L)]
            gs = [jnp.where(g > 0.5, sel[l], 0.0) for l, g in enumerate(gam)]
            for i in range(I):
                for l in range(L):
                    cap[i] = cap[i] + row(_B_H + (j * I + i) * L + l) * gs[l]

        tbv = [plsc.load_gather(tb_v, [lax.iota(jnp.int32, 16),
                                       jnp.full((16,), l, jnp.int32)])
               for l in range(L)]
        x = jnp.zeros((16,), jnp.float32)
        for l in range(L):
            c = tbv[l]
            for i in range(I):
                c = c + cap[i] * row(_B_W + l * I + i)
            x = x + c * c

        yi = 0x5F3759DF - (plsc.bitcast(x, jnp.int32) >> 1)
        y = plsc.bitcast(yi, jnp.float32)
        for _ in range(3):
            y = y * (1.5 - 0.5 * x * y * y)
        p = x * y

        e = jnp.exp(p - jnp.max(p))
        o_v[...] = e / jnp.sum(e)
        pltpu.sync_copy(o_v, out_hbm)


@jax.jit
def kernel(state, constants, gammas, head_W, tail_W, tail_b):
    f32 = jnp.float32
    packed = pl.pallas_call(
        _pack_body,
        out_shape=jax.ShapeDtypeStruct((8, 128), f32),
    )(constants, gammas, head_W, tail_W)

    run = pl.kernel(
        _body,
        out_type=jax.ShapeDtypeStruct((M,), f32),
        mesh=plsc.VectorSubcoreMesh(core_axis_name="c", subcore_axis_name="s",
                                    num_cores=1, num_subcores=1),
        scratch_types=[
            pltpu.VMEM((8, 128), f32),
            pltpu.VMEM((1, W * L), f32),
            pltpu.VMEM((M, L), f32),
            pltpu.VMEM((M,), f32),
            pltpu.SemaphoreType.DMA,
        ],
        compiler_params=pltpu.CompilerParams(needs_layout_passes=False,
                                             skip_device_barrier=True),
    )
    return run(packed, state, tail_b)


# speedup vs baseline: 1.0024x; 1.0024x over previous
import jax
import jax.numpy as jnp
from jax import lax
from jax.experimental import pallas as pl
from jax.experimental.pallas import tpu as pltpu
from jax.experimental.pallas import tpu_sc as plsc

M, J, I, L, W = 16, 2, 3, 2, 9

_B_G = W * L
_B_T = _B_G + J * L
_B_H = _B_T + J * L
_B_W = _B_H + J * I * L
_B_B = _B_W + L * I


def _pack_body(st, con, gam, hw, tw, tb, o):
    def put(r, v):
        o[r % 8, pl.ds((r // 8) * 16, 16)] = v

    for w in range(W):
        for l in range(L):
            put(w * L + l, jnp.full((16,), st[0, w * L + l], jnp.float32))
    for j in range(J):
        for l in range(L):
            put(_B_G + j * L + l, gam[:, j + 1, l])
            put(_B_T + j * L + l, con[:, j, l])
    for j in range(J):
        for i in range(I):
            for l in range(L):
                put(_B_H + (j * I + i) * L + l, hw[:, j, i, l])
    for l in range(L):
        for i in range(I):
            put(_B_W + l * I + i, tw[:, l, i])
        put(_B_B + l, tb[:, l])


def _body(a_hbm, out_hbm, a_v, o_v):
    @pl.when((lax.axis_index("c") == 0) & (lax.axis_index("s") == 0))
    def _():
        pltpu.sync_copy(a_hbm, a_v)

        def row(r):
            return a_v[r % 8, pl.ds((r // 8) * 16, 16)]

        s = [[row(w * L + l) for l in range(L)] for w in range(W)]
        cap = [jnp.zeros((16,), jnp.float32) for _ in range(I)]
        for j in range(J):
            gam = [row(_B_G + j * L + l) for l in range(L)]
            sig = [1.0 / (1.0 + jnp.exp(-10.0 * (g - 0.5))) for g in gam]
            tem = [row(_B_T + j * L + l) for l in range(L)]
            best_q = None
            sel = [None] * L
            for w in range(W):
                d0 = tem[0] - s[w][0]
                q = sig[0] * d0 * d0
                for l in range(1, L):
                    dl = tem[l] - s[w][l]
                    q = q + sig[l] * dl * dl
                if best_q is None:
                    best_q = q
                    sel = list(s[w])
                else:
                    take = q < best_q
                    best_q = jnp.where(take, q, best_q)
                    sel = [jnp.where(take, s[w][l], sel[l]) for l in range(L)]
            gs = [jnp.where(g > 0.5, sel[l], 0.0) for l, g in enumerate(gam)]
            for i in range(I):
                for l in range(L):
                    cap[i] = cap[i] + row(_B_H + (j * I + i) * L + l) * gs[l]

        x = jnp.zeros((16,), jnp.float32)
        for l in range(L):
            c = row(_B_B + l)
            for i in range(I):
                c = c + cap[i] * row(_B_W + l * I + i)
            x = x + c * c

        yi = 0x5F3759DF - (plsc.bitcast(x, jnp.int32) >> 1)
        y = plsc.bitcast(yi, jnp.float32)
        for _ in range(3):
            y = y * (1.5 - 0.5 * x * y * y)
        p = x * y

        e = jnp.exp(p - jnp.max(p))
        o_v[...] = e / jnp.sum(e)
        pltpu.sync_copy(o_v, out_hbm)


@jax.jit
def kernel(state, constants, gammas, head_W, tail_W, tail_b):
    f32 = jnp.float32
    packed = pl.pallas_call(
        _pack_body,
        out_shape=jax.ShapeDtypeStruct((8, 128), f32),
    )(state, constants, gammas, head_W, tail_W, tail_b)

    run = pl.kernel(
        _body,
        out_type=jax.ShapeDtypeStruct((M,), f32),
        mesh=plsc.VectorSubcoreMesh(core_axis_name="c", subcore_axis_name="s",
                                    num_cores=1, num_subcores=1),
        scratch_types=[
            pltpu.VMEM((8, 128), f32),
            pltpu.VMEM((M,), f32),
        ],
        compiler_params=pltpu.CompilerParams(needs_layout_passes=False,
                                             skip_device_barrier=True),
    )
    return run(packed)
